# Initial kernel scaffold; baseline (speedup 1.0000x reference)
import jax
import jax.numpy as jnp
from jax.experimental import pallas as pl


def kernel(x, pe_weight):
    raise NotImplementedError("write your pallas kernel here")


# speedup vs baseline: 1.8062x; 1.8062x over previous
import jax
import jax.numpy as jnp
from jax.experimental import pallas as pl


_SBLK = 512


def _add_pe_kernel(x_ref, pe_ref, o_ref):
    o_ref[...] = x_ref[...] + pe_ref[...][None, :, :]


def kernel(x, pe_weight):
    B, S, D = x.shape
    grid = (S // _SBLK,)
    return pl.pallas_call(
        _add_pe_kernel,
        grid=grid,
        in_specs=[
            pl.BlockSpec((B, _SBLK, D), lambda i: (0, i, 0)),
            pl.BlockSpec((_SBLK, D), lambda i: (i, 0)),
        ],
        out_specs=pl.BlockSpec((B, _SBLK, D), lambda i: (0, i, 0)),
        out_shape=jax.ShapeDtypeStruct((B, S, D), x.dtype),
    )(x, pe_weight)


# speedup vs baseline: 1.8110x; 1.0027x over previous
import jax
import jax.numpy as jnp
from jax.experimental import pallas as pl


_SBLK = 1024


def _add_pe_kernel(x_ref, pe_ref, o_ref):
    o_ref[...] = x_ref[...] + pe_ref[...][None, :, :]


def kernel(x, pe_weight):
    B, S, D = x.shape
    grid = (S // _SBLK,)
    return pl.pallas_call(
        _add_pe_kernel,
        grid=grid,
        in_specs=[
            pl.BlockSpec((B, _SBLK, D), lambda i: (0, i, 0)),
            pl.BlockSpec((_SBLK, D), lambda i: (i, 0)),
        ],
        out_specs=pl.BlockSpec((B, _SBLK, D), lambda i: (0, i, 0)),
        out_shape=jax.ShapeDtypeStruct((B, S, D), x.dtype),
    )(x, pe_weight)
